# Initial kernel scaffold; baseline (speedup 1.0000x reference)
import jax
import jax.numpy as jnp
from jax.experimental import pallas as pl


def kernel(pos, batch, edge_src, edge_dst, fc1_1, fc2_1, Wmix1, Smix1, fc1_2, fc2_2, Wmix2, Smix2):
    raise NotImplementedError("write your pallas kernel here")


# speedup vs baseline: 1.2911x; 1.2911x over previous
import functools

import jax
import jax.numpy as jnp
from jax import lax
from jax.experimental import pallas as pl
from jax.experimental.pallas import tpu as pltpu
from jax.experimental.pallas import tpu_sc as plsc

N = 10000
E = 160000
NG = 64
NPAD = 10240
CHUNK = 128
NW = 32
CPW = 40
EPAD = NW * CPW * CHUNK
BE = 2048
INV_SQRT_NEI = float(3.8 ** -0.5)
SQRT3 = float(3.0 ** 0.5)


def _mesh():
    return plsc.VectorSubcoreMesh(core_axis_name="c", subcore_axis_name="s")


def _make_gather2(T):
    out_sds = (jax.ShapeDtypeStruct((EPAD, 128), jnp.float32),) * 2

    @functools.partial(
        pl.kernel, out_type=out_sds, mesh=_mesh(),
        scratch_types=[pltpu.VMEM((CPW, CHUNK), jnp.int32),
                       pltpu.VMEM((CPW, CHUNK), jnp.int32),
                       pltpu.VMEM((CHUNK, 128), jnp.float32),
                       pltpu.VMEM((CHUNK, 128), jnp.float32)],
    )
    def k(table, idxa, idxb, outa, outb, ia_v, ib_v, ra_v, rb_v):
        wid = lax.axis_index("s") * 2 + lax.axis_index("c")
        base = wid * (CPW * CHUNK)
        pltpu.sync_copy(idxa.at[wid], ia_v)
        pltpu.sync_copy(idxb.at[wid], ib_v)

        def body(j, c):
            pltpu.sync_copy(table.at[ia_v.at[j]], ra_v)
            pltpu.sync_copy(ra_v, outa.at[pl.ds(base + j * CHUNK, CHUNK)])
            pltpu.sync_copy(table.at[ib_v.at[j]], rb_v)
            pltpu.sync_copy(rb_v, outb.at[pl.ds(base + j * CHUNK, CHUNK)])
            return c

        lax.fori_loop(0, CPW, body, 0)

    return k


def _make_gather1(T):
    @functools.partial(
        pl.kernel, out_type=jax.ShapeDtypeStruct((EPAD, 128), jnp.float32),
        mesh=_mesh(),
        scratch_types=[pltpu.VMEM((CPW, CHUNK), jnp.int32),
                       pltpu.VMEM((CHUNK, 128), jnp.float32)],
    )
    def k(table, idx, out, i_v, r_v):
        wid = lax.axis_index("s") * 2 + lax.axis_index("c")
        base = wid * (CPW * CHUNK)
        pltpu.sync_copy(idx.at[wid], i_v)

        def body(j, c):
            pltpu.sync_copy(table.at[i_v.at[j]], r_v)
            pltpu.sync_copy(r_v, out.at[pl.ds(base + j * CHUNK, CHUNK)])
            return c

        lax.fori_loop(0, CPW, body, 0)

    return k


def _make_scatter():
    RPT = NPAD // 16

    @functools.partial(
        pl.kernel, out_type=jax.ShapeDtypeStruct((2 * NPAD, 128), jnp.float32),
        mesh=_mesh(),
        scratch_types=[pltpu.VMEM((CPW, CHUNK), jnp.int32),
                       pltpu.VMEM((CHUNK, 128), jnp.float32),
                       pltpu.VMEM_SHARED((NPAD, 128), jnp.float32)],
    )
    def k(data, idx, zeros, out, i_v, r_v, acc):
        cid = lax.axis_index("c")
        sid = lax.axis_index("s")
        wid = sid * 2 + cid
        base = wid * (CPW * CHUNK)
        pltpu.sync_copy(zeros.at[pl.ds(sid * RPT, RPT)],
                        acc.at[pl.ds(sid * RPT, RPT)])
        pltpu.sync_copy(idx.at[wid], i_v)
        plsc.subcore_barrier()

        def body(j, c):
            pltpu.sync_copy(data.at[pl.ds(base + j * CHUNK, CHUNK)], r_v)
            pltpu.sync_copy(r_v, acc.at[i_v.at[j]], add=True)
            return c

        lax.fori_loop(0, CPW, body, 0)
        plsc.subcore_barrier()
        pltpu.sync_copy(acc.at[pl.ds(sid * RPT, RPT)],
                        out.at[pl.ds(cid * NPAD + sid * RPT, RPT)])

    return k


def _edge_body(psrc_ref, pdst_ref, f11_ref, f21_ref, f12_ref, f22_ref,
               ew_ref, w2_ref):
    ev = psrc_ref[...] - pdst_ref[...]
    x = ev[:, 0:1]
    y = ev[:, 1:2]
    z = ev[:, 2:3]
    r = jnp.sqrt(x * x + y * y + z * z)
    inv = 1.0 / (r + 1e-9)
    ux, uy, uz = x * inv, y * inv, z * inv
    s5 = 5.0 ** 0.5
    s15 = 15.0 ** 0.5
    c1 = (35.0 / 8.0) ** 0.5
    c2 = 105.0 ** 0.5
    c3 = (21.0 / 8.0) ** 0.5
    c4 = (7.0 ** 0.5) / 2.0
    one = jnp.ones_like(ux)
    sh = jnp.concatenate([
        one,
        SQRT3 * ux, SQRT3 * uy, SQRT3 * uz,
        s15 * ux * uy,
        s15 * uy * uz,
        0.5 * s5 * (3.0 * uz * uz - 1.0),
        s15 * ux * uz,
        0.5 * s15 * (ux * ux - uy * uy),
        c1 * uy * (3.0 * ux * ux - uy * uy),
        c2 * ux * uy * uz,
        c3 * uy * (5.0 * uz * uz - 1.0),
        c4 * uz * (5.0 * uz * uz - 3.0),
        c3 * ux * (5.0 * uz * uz - 1.0),
        0.5 * c2 * uz * (ux * ux - uy * uy),
        c1 * ux * (ux * ux - 3.0 * uy * uy),
    ], axis=1)
    coli = lax.broadcasted_iota(jnp.int32, (BE, 16), 1)
    col = coli.astype(jnp.float32)
    d = (r - (1.0 + 0.5 * col)) * 2.0
    embp = jnp.where(coli < 3, jnp.exp(-d * d) * SQRT3, 0.0)
    h1 = jnp.maximum(jnp.dot(embp, f11_ref[...],
                             preferred_element_type=jnp.float32), 0.0)
    w1 = jnp.dot(h1, f21_ref[...],
                 preferred_element_type=jnp.float32) * (1.0 / (SQRT3 * 16.0))
    h2 = jnp.maximum(jnp.dot(embp, f12_ref[...],
                             preferred_element_type=jnp.float32), 0.0)
    w2_ref[...] = jnp.dot(h2, f22_ref[...],
                          preferred_element_type=jnp.float32) * (1.0 / (SQRT3 * 16.0))
    ew_ref[...] = jnp.concatenate(
        [sh, w1, jnp.zeros((BE, 96), jnp.float32)], axis=1)


def _msg1_body(x0_ref, ew_ref, wmix_ref, smix_ref, t1_ref, t2_ref):
    ew = ew_ref[...]
    xw = x0_ref[...][:, :16] * INV_SQRT_NEI * ew[:, 16:32]
    m1 = (jnp.dot(xw, wmix_ref[...], preferred_element_type=jnp.float32)
          * jnp.dot(ew[:, :16], smix_ref[...],
                    preferred_element_type=jnp.float32))
    t1_ref[...] = m1[:, :128]
    t2_ref[...] = jnp.concatenate(
        [m1[:, 128:], jnp.zeros((BE, 96), jnp.float32)], axis=1)


def _msg2_body(x1_ref, w2_ref, ew_ref, wmix_ref, smix_ref, m_ref):
    xw = x1_ref[...] * w2_ref[...]
    m2 = (jnp.dot(xw, wmix_ref[...], preferred_element_type=jnp.float32)
          * jnp.dot(ew_ref[...][:, :16], smix_ref[...],
                    preferred_element_type=jnp.float32))
    m_ref[...] = jnp.concatenate(
        [m2, jnp.zeros((BE, 112), jnp.float32)], axis=1)


def _add2_body(a_ref, b_ref, o_ref):
    o_ref[...] = a_ref[...] + b_ref[...]


def _gate_body(y1a_ref, y1b_ref, y2a_ref, y2b_ref, x1_ref):
    p = (y1a_ref[...] + y1b_ref[...]) * INV_SQRT_NEI
    q = (y2a_ref[...] + y2b_ref[...]) * INV_SQRT_NEI
    xx = jnp.concatenate([p, q[:, :32]], axis=1)
    scalars = jnp.concatenate(
        [jnp.maximum(xx[:, :16], 0.0), jnp.abs(xx[:, 16:32])], axis=1)
    gates = jnp.concatenate([
        jnp.maximum(xx[:, 32:40], 0.0), jnp.tanh(xx[:, 40:48]),
        jnp.maximum(xx[:, 48:56], 0.0), jnp.tanh(xx[:, 56:64]),
    ], axis=1)
    g_row = lax.broadcasted_iota(jnp.int32, (32, 96), 0)
    j_col = lax.broadcasted_iota(jnp.int32, (32, 96), 1)
    expand = jnp.where(g_row == j_col // 3, 1.0, 0.0)
    gates96 = jnp.dot(gates, expand, preferred_element_type=jnp.float32)
    x1_ref[...] = jnp.concatenate([scalars, xx[:, 64:160] * gates96], axis=1)


def _graph_body(ya_ref, yb_ref, batch_ref, out_ref):
    i = pl.program_id(0)

    @pl.when(i == 0)
    def _():
        out_ref[...] = jnp.zeros_like(out_ref)

    y = (ya_ref[0] + yb_ref[0]) * (INV_SQRT_NEI * 0.5)
    b = batch_ref[...]
    gcol = lax.broadcasted_iota(jnp.int32, (b.shape[0], NG), 1).astype(jnp.float32)
    onehot = jnp.where(b == gcol, 1.0, 0.0)
    out_ref[...] += lax.dot_general(onehot, y, (((0,), (0,)), ((), ())),
                                    preferred_element_type=jnp.float32)


def _wfull(a):
    return pl.BlockSpec(a.shape, lambda i: (0,) * a.ndim)


def _edge_precompute(psrc, pdst, f11, f21, f12, f22):
    grid = EPAD // BE
    eb = pl.BlockSpec((BE, 128), lambda i: (i, 0))
    return pl.pallas_call(
        _edge_body,
        grid=(grid,),
        in_specs=[eb, eb, _wfull(f11), _wfull(f21), _wfull(f12), _wfull(f22)],
        out_specs=[eb, eb],
        out_shape=[jax.ShapeDtypeStruct((EPAD, 128), jnp.float32),
                   jax.ShapeDtypeStruct((EPAD, 128), jnp.float32)],
    )(psrc, pdst, f11, f21, f12, f22)


def _messages1(x0src, ew, wmix, smix):
    grid = EPAD // BE
    eb = pl.BlockSpec((BE, 128), lambda i: (i, 0))
    return pl.pallas_call(
        _msg1_body,
        grid=(grid,),
        in_specs=[eb, eb, _wfull(wmix), _wfull(smix)],
        out_specs=[eb, eb],
        out_shape=[jax.ShapeDtypeStruct((EPAD, 128), jnp.float32),
                   jax.ShapeDtypeStruct((EPAD, 128), jnp.float32)],
    )(x0src, ew, wmix, smix)


def _messages2(x1src, w2, ew, wmix, smix):
    grid = EPAD // BE
    eb = pl.BlockSpec((BE, 128), lambda i: (i, 0))
    return pl.pallas_call(
        _msg2_body,
        grid=(grid,),
        in_specs=[eb, eb, eb, _wfull(wmix), _wfull(smix)],
        out_specs=eb,
        out_shape=jax.ShapeDtypeStruct((EPAD, 128), jnp.float32),
    )(x1src, w2, ew, wmix, smix)


def _add2(pair):
    BN = 1024
    grid = NPAD // BN
    return pl.pallas_call(
        _add2_body,
        grid=(grid,),
        in_specs=[pl.BlockSpec((BN, 128), lambda i: (i, 0)),
                  pl.BlockSpec((BN, 128), lambda i: (i + NPAD // BN, 0))],
        out_specs=pl.BlockSpec((BN, 128), lambda i: (i, 0)),
        out_shape=jax.ShapeDtypeStruct((NPAD, 128), jnp.float32),
    )(pair, pair)


def _gate(y1p, y2p):
    BN = 1024
    grid = NPAD // BN
    lo = pl.BlockSpec((BN, 128), lambda i: (i, 0))
    hi = pl.BlockSpec((BN, 128), lambda i: (i + NPAD // BN, 0))
    return pl.pallas_call(
        _gate_body,
        grid=(grid,),
        in_specs=[lo, hi, lo, hi],
        out_specs=lo,
        out_shape=jax.ShapeDtypeStruct((NPAD, 128), jnp.float32),
    )(y1p, y1p, y2p, y2p)


def _graph_reduce(y2pair, batch_f):
    BN = 2000
    grid = N // BN
    return pl.pallas_call(
        _graph_body,
        grid=(grid,),
        in_specs=[pl.BlockSpec((1, BN, 128), lambda i: (0, i, 0)),
                  pl.BlockSpec((1, BN, 128), lambda i: (1, i, 0)),
                  pl.BlockSpec((BN, 1), lambda i: (i, 0))],
        out_specs=pl.BlockSpec((NG, 128), lambda i: (0, 0)),
        out_shape=jax.ShapeDtypeStruct((NG, 128), jnp.float32),
    )(y2pair, y2pair, batch_f)


def kernel(pos, batch, edge_src, edge_dst, fc1_1, fc2_1, Wmix1, Smix1,
           fc1_2, fc2_2, Wmix2, Smix2):
    f32 = jnp.float32
    es = edge_src.astype(jnp.int32)
    ed = edge_dst.astype(jnp.int32)
    padn = EPAD - E
    es_g = jnp.concatenate([es, jnp.zeros((padn,), jnp.int32)])
    ed_g = jnp.concatenate([ed, jnp.zeros((padn,), jnp.int32)])
    ed_s = jnp.concatenate([ed, jnp.full((padn,), N, jnp.int32)])
    i_src = es_g.reshape(NW, CPW, CHUNK)
    i_dst = ed_g.reshape(NW, CPW, CHUNK)
    i_dst_s = ed_s.reshape(NW, CPW, CHUNK)

    pos128 = jnp.pad(pos.astype(f32), ((0, 0), (0, 125)))
    f11 = jnp.pad(fc1_1.astype(f32), ((0, 13), (0, 0)))
    f12 = jnp.pad(fc1_2.astype(f32), ((0, 13), (0, 0)))
    wmix2p = jnp.pad(Wmix2.astype(f32), ((0, 0), (0, 9)))
    smix2p = jnp.pad(Smix2.astype(f32), ((0, 0), (0, 9)))
    batch_f = batch.astype(f32)[:, None]
    z128 = jnp.zeros((NPAD, 128), f32)

    psrc, pdst = _make_gather2(N)(pos128, i_src, i_dst)
    ew, w2 = _edge_precompute(psrc, pdst, f11, fc2_1.astype(f32), f12,
                              fc2_2.astype(f32))
    x0p = _make_scatter()(ew, i_dst_s, z128)
    x0c = _add2(x0p)
    x0src = _make_gather1(NPAD)(x0c, i_src)
    t1, t2 = _messages1(x0src, ew, Wmix1.astype(f32), Smix1.astype(f32))
    y1p = _make_scatter()(t1, i_dst_s, z128)
    y2p_ = _make_scatter()(t2, i_dst_s, z128)
    x1 = _gate(y1p, y2p_)
    x1src = _make_gather1(NPAD)(x1, i_src)
    m2 = _messages2(x1src, w2, ew, wmix2p, smix2p)
    y2p = _make_scatter()(m2, i_dst_s, z128)
    out = _graph_reduce(y2p.reshape(2, NPAD, 128), batch_f)
    return out[:, :7]


# speedup vs baseline: 1.3291x; 1.0294x over previous
import functools

import jax
import jax.numpy as jnp
from jax import lax
from jax.experimental import pallas as pl
from jax.experimental.pallas import tpu as pltpu
from jax.experimental.pallas import tpu_sc as plsc

N = 10000
E = 160000
NG = 64
NPAD = 10240
CHUNK = 128
NW = 32
CPW = 40
EPAD = NW * CPW * CHUNK
BE = 2048
INV_SQRT_NEI = float(3.8 ** -0.5)
SQRT3 = float(3.0 ** 0.5)


def _mesh():
    return plsc.VectorSubcoreMesh(core_axis_name="c", subcore_axis_name="s")


NB = 4
NROUNDS = CPW // NB


def _make_gather1(T):
    scratch = ([pltpu.VMEM((CPW, CHUNK), jnp.int32)]
               + [pltpu.VMEM((CHUNK, 128), jnp.float32)] * NB
               + [pltpu.SemaphoreType.DMA] * (2 * NB))

    @functools.partial(
        pl.kernel, out_type=jax.ShapeDtypeStruct((EPAD, 128), jnp.float32),
        mesh=_mesh(), scratch_types=scratch,
    )
    def k(table, idx, out, i_v, *rs):
        bufs = rs[:NB]
        gs = rs[NB:2 * NB]
        ws = rs[2 * NB:]
        wid = lax.axis_index("s") * 2 + lax.axis_index("c")
        base = wid * (CPW * CHUNK)
        pltpu.sync_copy(idx.at[wid], i_v)
        for b in range(NB):
            pltpu.async_copy(table.at[i_v.at[b]], bufs[b], gs[b])

        def rnd(r, c):
            for b in range(NB):
                j = r * NB + b
                pltpu.make_async_copy(table.at[i_v.at[j]], bufs[b],
                                      gs[b]).wait()
                pltpu.async_copy(bufs[b],
                                 out.at[pl.ds(base + j * CHUNK, CHUNK)],
                                 ws[b])
            for b in range(NB):
                jn = (r + 1) * NB + b

                @pl.when(jn < CPW)
                def _():
                    pltpu.make_async_copy(
                        bufs[b], out.at[pl.ds(base, CHUNK)], ws[b]).wait()
                    pltpu.async_copy(table.at[i_v.at[jn]], bufs[b], gs[b])
            return c

        lax.fori_loop(0, NROUNDS, rnd, 0)
        for b in range(NB):
            pltpu.make_async_copy(bufs[b], out.at[pl.ds(base, CHUNK)],
                                  ws[b]).wait()

    return k


def _make_scatter():
    RPT = NPAD // 16
    NBS = 2
    scratch = ([pltpu.VMEM((CPW, CHUNK), jnp.int32)]
               + [pltpu.VMEM((CHUNK, 128), jnp.float32)] * NBS
               + [pltpu.SemaphoreType.DMA] * (2 * NBS))

    @functools.partial(
        pl.kernel, out_type=jax.ShapeDtypeStruct((2 * NPAD, 128), jnp.float32),
        mesh=_mesh(),
        scratch_types=scratch + [pltpu.VMEM_SHARED((NPAD, 128), jnp.float32)],
    )
    def k(data, idx, zeros, out, i_v, *rs):
        bufs = rs[:NBS]
        ls = rs[NBS:2 * NBS]
        ss = rs[2 * NBS:3 * NBS]
        acc = rs[3 * NBS]
        cid = lax.axis_index("c")
        sid = lax.axis_index("s")
        wid = sid * 2 + cid
        base = wid * (CPW * CHUNK)
        pltpu.sync_copy(zeros.at[pl.ds(sid * RPT, RPT)],
                        acc.at[pl.ds(sid * RPT, RPT)])
        pltpu.sync_copy(idx.at[wid], i_v)
        plsc.subcore_barrier()
        for b in range(NBS):
            pltpu.async_copy(data.at[pl.ds(base + b * CHUNK, CHUNK)],
                             bufs[b], ls[b])

        def rnd(r, c):
            for b in range(NBS):
                j = r * NBS + b
                pltpu.make_async_copy(
                    data.at[pl.ds(base, CHUNK)], bufs[b], ls[b]).wait()
                pltpu.async_copy(bufs[b], acc.at[i_v.at[j]], ss[b], add=True)
            for b in range(NBS):
                jn = (r + 1) * NBS + b

                @pl.when(jn < CPW)
                def _():
                    pltpu.make_async_copy(bufs[b], acc.at[i_v.at[0]],
                                          ss[b]).wait()
                    pltpu.async_copy(data.at[pl.ds(base + jn * CHUNK, CHUNK)],
                                     bufs[b], ls[b])
            return c

        lax.fori_loop(0, CPW // NBS, rnd, 0)
        for b in range(NBS):
            pltpu.make_async_copy(bufs[b], acc.at[i_v.at[0]], ss[b]).wait()
        plsc.subcore_barrier()
        pltpu.sync_copy(acc.at[pl.ds(sid * RPT, RPT)],
                        out.at[pl.ds(cid * NPAD + sid * RPT, RPT)])

    return k


def _edge_body(psrc_ref, pdst_ref, f11_ref, f21_ref, f12_ref, f22_ref,
               ew_ref, w2_ref):
    ev = psrc_ref[...] - pdst_ref[...]
    x = ev[:, 0:1]
    y = ev[:, 1:2]
    z = ev[:, 2:3]
    r = jnp.sqrt(x * x + y * y + z * z)
    inv = 1.0 / (r + 1e-9)
    ux, uy, uz = x * inv, y * inv, z * inv
    s5 = 5.0 ** 0.5
    s15 = 15.0 ** 0.5
    c1 = (35.0 / 8.0) ** 0.5
    c2 = 105.0 ** 0.5
    c3 = (21.0 / 8.0) ** 0.5
    c4 = (7.0 ** 0.5) / 2.0
    one = jnp.ones_like(ux)
    sh = jnp.concatenate([
        one,
        SQRT3 * ux, SQRT3 * uy, SQRT3 * uz,
        s15 * ux * uy,
        s15 * uy * uz,
        0.5 * s5 * (3.0 * uz * uz - 1.0),
        s15 * ux * uz,
        0.5 * s15 * (ux * ux - uy * uy),
        c1 * uy * (3.0 * ux * ux - uy * uy),
        c2 * ux * uy * uz,
        c3 * uy * (5.0 * uz * uz - 1.0),
        c4 * uz * (5.0 * uz * uz - 3.0),
        c3 * ux * (5.0 * uz * uz - 1.0),
        0.5 * c2 * uz * (ux * ux - uy * uy),
        c1 * ux * (ux * ux - 3.0 * uy * uy),
    ], axis=1)
    coli = lax.broadcasted_iota(jnp.int32, (BE, 16), 1)
    col = coli.astype(jnp.float32)
    d = (r - (1.0 + 0.5 * col)) * 2.0
    embp = jnp.where(coli < 3, jnp.exp(-d * d) * SQRT3, 0.0)
    h1 = jnp.maximum(jnp.dot(embp, f11_ref[...],
                             preferred_element_type=jnp.float32), 0.0)
    w1 = jnp.dot(h1, f21_ref[...],
                 preferred_element_type=jnp.float32) * (1.0 / (SQRT3 * 16.0))
    h2 = jnp.maximum(jnp.dot(embp, f12_ref[...],
                             preferred_element_type=jnp.float32), 0.0)
    w2_ref[...] = jnp.dot(h2, f22_ref[...],
                          preferred_element_type=jnp.float32) * (1.0 / (SQRT3 * 16.0))
    ew_ref[...] = jnp.concatenate(
        [sh, w1, jnp.zeros((BE, 96), jnp.float32)], axis=1)


def _msg1_body(x0_ref, ew_ref, wmix_ref, smix_ref, t1_ref, t2_ref):
    ew = ew_ref[...]
    xw = x0_ref[...][:, :16] * INV_SQRT_NEI * ew[:, 16:32]
    m1 = (jnp.dot(xw, wmix_ref[...], preferred_element_type=jnp.float32)
          * jnp.dot(ew[:, :16], smix_ref[...],
                    preferred_element_type=jnp.float32))
    t1_ref[...] = m1[:, :128]
    t2_ref[...] = jnp.concatenate(
        [m1[:, 128:], jnp.zeros((BE, 96), jnp.float32)], axis=1)


def _msg2_body(x1_ref, w2_ref, ew_ref, wmix_ref, smix_ref, m_ref):
    xw = x1_ref[...] * w2_ref[...]
    m2 = (jnp.dot(xw, wmix_ref[...], preferred_element_type=jnp.float32)
          * jnp.dot(ew_ref[...][:, :16], smix_ref[...],
                    preferred_element_type=jnp.float32))
    m_ref[...] = jnp.concatenate(
        [m2, jnp.zeros((BE, 112), jnp.float32)], axis=1)


def _add2_body(a_ref, b_ref, o_ref):
    o_ref[...] = a_ref[...] + b_ref[...]


def _gate_body(y1a_ref, y1b_ref, y2a_ref, y2b_ref, x1_ref):
    p = (y1a_ref[...] + y1b_ref[...]) * INV_SQRT_NEI
    q = (y2a_ref[...] + y2b_ref[...]) * INV_SQRT_NEI
    xx = jnp.concatenate([p, q[:, :32]], axis=1)
    scalars = jnp.concatenate(
        [jnp.maximum(xx[:, :16], 0.0), jnp.abs(xx[:, 16:32])], axis=1)
    gates = jnp.concatenate([
        jnp.maximum(xx[:, 32:40], 0.0), jnp.tanh(xx[:, 40:48]),
        jnp.maximum(xx[:, 48:56], 0.0), jnp.tanh(xx[:, 56:64]),
    ], axis=1)
    g_row = lax.broadcasted_iota(jnp.int32, (32, 96), 0)
    j_col = lax.broadcasted_iota(jnp.int32, (32, 96), 1)
    expand = jnp.where(g_row == j_col // 3, 1.0, 0.0)
    gates96 = jnp.dot(gates, expand, preferred_element_type=jnp.float32)
    x1_ref[...] = jnp.concatenate([scalars, xx[:, 64:160] * gates96], axis=1)


def _graph_body(ya_ref, yb_ref, batch_ref, out_ref):
    i = pl.program_id(0)

    @pl.when(i == 0)
    def _():
        out_ref[...] = jnp.zeros_like(out_ref)

    y = (ya_ref[0] + yb_ref[0]) * (INV_SQRT_NEI * 0.5)
    b = batch_ref[...]
    gcol = lax.broadcasted_iota(jnp.int32, (b.shape[0], NG), 1).astype(jnp.float32)
    onehot = jnp.where(b == gcol, 1.0, 0.0)
    out_ref[...] += lax.dot_general(onehot, y, (((0,), (0,)), ((), ())),
                                    preferred_element_type=jnp.float32)


def _wfull(a):
    return pl.BlockSpec(a.shape, lambda i: (0,) * a.ndim)


def _edge_precompute(psrc, pdst, f11, f21, f12, f22):
    grid = EPAD // BE
    eb = pl.BlockSpec((BE, 128), lambda i: (i, 0))
    return pl.pallas_call(
        _edge_body,
        grid=(grid,),
        in_specs=[eb, eb, _wfull(f11), _wfull(f21), _wfull(f12), _wfull(f22)],
        out_specs=[eb, eb],
        out_shape=[jax.ShapeDtypeStruct((EPAD, 128), jnp.float32),
                   jax.ShapeDtypeStruct((EPAD, 128), jnp.float32)],
    )(psrc, pdst, f11, f21, f12, f22)


def _messages1(x0src, ew, wmix, smix):
    grid = EPAD // BE
    eb = pl.BlockSpec((BE, 128), lambda i: (i, 0))
    return pl.pallas_call(
        _msg1_body,
        grid=(grid,),
        in_specs=[eb, eb, _wfull(wmix), _wfull(smix)],
        out_specs=[eb, eb],
        out_shape=[jax.ShapeDtypeStruct((EPAD, 128), jnp.float32),
                   jax.ShapeDtypeStruct((EPAD, 128), jnp.float32)],
    )(x0src, ew, wmix, smix)


def _messages2(x1src, w2, ew, wmix, smix):
    grid = EPAD // BE
    eb = pl.BlockSpec((BE, 128), lambda i: (i, 0))
    return pl.pallas_call(
        _msg2_body,
        grid=(grid,),
        in_specs=[eb, eb, eb, _wfull(wmix), _wfull(smix)],
        out_specs=eb,
        out_shape=jax.ShapeDtypeStruct((EPAD, 128), jnp.float32),
    )(x1src, w2, ew, wmix, smix)


def _add2(pair):
    BN = 1024
    grid = NPAD // BN
    return pl.pallas_call(
        _add2_body,
        grid=(grid,),
        in_specs=[pl.BlockSpec((BN, 128), lambda i: (i, 0)),
                  pl.BlockSpec((BN, 128), lambda i: (i + NPAD // BN, 0))],
        out_specs=pl.BlockSpec((BN, 128), lambda i: (i, 0)),
        out_shape=jax.ShapeDtypeStruct((NPAD, 128), jnp.float32),
    )(pair, pair)


def _gate(y1p, y2p):
    BN = 1024
    grid = NPAD // BN
    lo = pl.BlockSpec((BN, 128), lambda i: (i, 0))
    hi = pl.BlockSpec((BN, 128), lambda i: (i + NPAD // BN, 0))
    return pl.pallas_call(
        _gate_body,
        grid=(grid,),
        in_specs=[lo, hi, lo, hi],
        out_specs=lo,
        out_shape=jax.ShapeDtypeStruct((NPAD, 128), jnp.float32),
    )(y1p, y1p, y2p, y2p)


def _graph_reduce(y2pair, batch_f):
    BN = 2000
    grid = N // BN
    return pl.pallas_call(
        _graph_body,
        grid=(grid,),
        in_specs=[pl.BlockSpec((1, BN, 128), lambda i: (0, i, 0)),
                  pl.BlockSpec((1, BN, 128), lambda i: (1, i, 0)),
                  pl.BlockSpec((BN, 1), lambda i: (i, 0))],
        out_specs=pl.BlockSpec((NG, 128), lambda i: (0, 0)),
        out_shape=jax.ShapeDtypeStruct((NG, 128), jnp.float32),
    )(y2pair, y2pair, batch_f)


def kernel(pos, batch, edge_src, edge_dst, fc1_1, fc2_1, Wmix1, Smix1,
           fc1_2, fc2_2, Wmix2, Smix2):
    f32 = jnp.float32
    es = edge_src.astype(jnp.int32)
    ed = edge_dst.astype(jnp.int32)
    padn = EPAD - E
    es_g = jnp.concatenate([es, jnp.zeros((padn,), jnp.int32)])
    ed_g = jnp.concatenate([ed, jnp.zeros((padn,), jnp.int32)])
    ed_s = jnp.concatenate([ed, jnp.full((padn,), N, jnp.int32)])
    i_src = es_g.reshape(NW, CPW, CHUNK)
    i_dst = ed_g.reshape(NW, CPW, CHUNK)
    i_dst_s = ed_s.reshape(NW, CPW, CHUNK)

    pos128 = jnp.pad(pos.astype(f32), ((0, 0), (0, 125)))
    f11 = jnp.pad(fc1_1.astype(f32), ((0, 13), (0, 0)))
    f12 = jnp.pad(fc1_2.astype(f32), ((0, 13), (0, 0)))
    wmix2p = jnp.pad(Wmix2.astype(f32), ((0, 0), (0, 9)))
    smix2p = jnp.pad(Smix2.astype(f32), ((0, 0), (0, 9)))
    batch_f = batch.astype(f32)[:, None]
    z128 = jnp.zeros((NPAD, 128), f32)

    psrc = _make_gather1(N)(pos128, i_src)
    pdst = _make_gather1(N)(pos128, i_dst)
    ew, w2 = _edge_precompute(psrc, pdst, f11, fc2_1.astype(f32), f12,
                              fc2_2.astype(f32))
    x0p = _make_scatter()(ew, i_dst_s, z128)
    x0c = _add2(x0p)
    x0src = _make_gather1(NPAD)(x0c, i_src)
    t1, t2 = _messages1(x0src, ew, Wmix1.astype(f32), Smix1.astype(f32))
    y1p = _make_scatter()(t1, i_dst_s, z128)
    y2p_ = _make_scatter()(t2, i_dst_s, z128)
    x1 = _gate(y1p, y2p_)
    x1src = _make_gather1(NPAD)(x1, i_src)
    m2 = _messages2(x1src, w2, ew, wmix2p, smix2p)
    y2p = _make_scatter()(m2, i_dst_s, z128)
    out = _graph_reduce(y2p.reshape(2, NPAD, 128), batch_f)
    return out[:, :7]


# speedup vs baseline: 2.0796x; 1.5646x over previous
import functools

import jax
import jax.numpy as jnp
from jax import lax
from jax.experimental import pallas as pl
from jax.experimental.pallas import tpu as pltpu
from jax.experimental.pallas import tpu_sc as plsc

N = 10000
E = 160000
NG = 64
NPAD = 10240
CHUNK = 128
NW = 32
CPW = 40
EPAD = NW * CPW * CHUNK
BE = 2048
INV_SQRT_NEI = float(3.8 ** -0.5)
SQRT3 = float(3.0 ** 0.5)


def _mesh():
    return plsc.VectorSubcoreMesh(core_axis_name="c", subcore_axis_name="s")


NB = 4
NROUNDS = CPW // NB


def _make_gather1(T):
    NBG = 2
    TPS = T // 16
    scratch = ([pltpu.VMEM((CPW, CHUNK), jnp.int32)]
               + [pltpu.VMEM((CHUNK, 128), jnp.float32)] * NBG
               + [pltpu.SemaphoreType.DMA] * (2 * NBG)
               + [pltpu.VMEM_SHARED((T, 128), jnp.float32)])

    @functools.partial(
        pl.kernel, out_type=jax.ShapeDtypeStruct((EPAD, 128), jnp.float32),
        mesh=_mesh(), scratch_types=scratch,
    )
    def k(table, idx, out, i_v, *rs):
        bufs = rs[:NBG]
        gs = rs[NBG:2 * NBG]
        ws = rs[2 * NBG:3 * NBG]
        stab = rs[3 * NBG]
        sid = lax.axis_index("s")
        wid = sid * 2 + lax.axis_index("c")
        base = wid * (CPW * CHUNK)
        pltpu.sync_copy(table.at[pl.ds(sid * TPS, TPS)],
                        stab.at[pl.ds(sid * TPS, TPS)])
        pltpu.sync_copy(idx.at[wid], i_v)
        plsc.subcore_barrier()
        for b in range(NBG):
            pltpu.async_copy(stab.at[i_v.at[b]], bufs[b], gs[b])

        def rnd(r, c):
            for b in range(NBG):
                j = r * NBG + b
                pltpu.make_async_copy(stab.at[i_v.at[j]], bufs[b],
                                      gs[b]).wait()
                pltpu.async_copy(bufs[b],
                                 out.at[pl.ds(base + j * CHUNK, CHUNK)],
                                 ws[b])
            for b in range(NBG):
                jn = (r + 1) * NBG + b

                @pl.when(jn < CPW)
                def _():
                    pltpu.make_async_copy(
                        bufs[b], out.at[pl.ds(base, CHUNK)], ws[b]).wait()
                    pltpu.async_copy(stab.at[i_v.at[jn]], bufs[b], gs[b])
            return c

        lax.fori_loop(0, CPW // NBG, rnd, 0)
        for b in range(NBG):
            pltpu.make_async_copy(bufs[b], out.at[pl.ds(base, CHUNK)],
                                  ws[b]).wait()

    return k


def _make_scatter():
    RPT = NPAD // 16
    NBS = 2
    scratch = ([pltpu.VMEM((CPW, CHUNK), jnp.int32)]
               + [pltpu.VMEM((CHUNK, 128), jnp.float32)] * NBS
               + [pltpu.SemaphoreType.DMA] * (2 * NBS))

    @functools.partial(
        pl.kernel, out_type=jax.ShapeDtypeStruct((2 * NPAD, 128), jnp.float32),
        mesh=_mesh(),
        scratch_types=scratch + [pltpu.VMEM_SHARED((NPAD, 128), jnp.float32)],
    )
    def k(data, idx, zeros, out, i_v, *rs):
        bufs = rs[:NBS]
        ls = rs[NBS:2 * NBS]
        ss = rs[2 * NBS:3 * NBS]
        acc = rs[3 * NBS]
        cid = lax.axis_index("c")
        sid = lax.axis_index("s")
        wid = sid * 2 + cid
        base = wid * (CPW * CHUNK)
        pltpu.sync_copy(zeros.at[pl.ds(sid * RPT, RPT)],
                        acc.at[pl.ds(sid * RPT, RPT)])
        pltpu.sync_copy(idx.at[wid], i_v)
        plsc.subcore_barrier()
        for b in range(NBS):
            pltpu.async_copy(data.at[pl.ds(base + b * CHUNK, CHUNK)],
                             bufs[b], ls[b])

        def rnd(r, c):
            for b in range(NBS):
                j = r * NBS + b
                pltpu.make_async_copy(
                    data.at[pl.ds(base, CHUNK)], bufs[b], ls[b]).wait()
                pltpu.async_copy(bufs[b], acc.at[i_v.at[j]], ss[b], add=True)
            for b in range(NBS):
                jn = (r + 1) * NBS + b

                @pl.when(jn < CPW)
                def _():
                    pltpu.make_async_copy(bufs[b], acc.at[i_v.at[0]],
                                          ss[b]).wait()
                    pltpu.async_copy(data.at[pl.ds(base + jn * CHUNK, CHUNK)],
                                     bufs[b], ls[b])
            return c

        lax.fori_loop(0, CPW // NBS, rnd, 0)
        for b in range(NBS):
            pltpu.make_async_copy(bufs[b], acc.at[i_v.at[0]], ss[b]).wait()
        plsc.subcore_barrier()
        pltpu.sync_copy(acc.at[pl.ds(sid * RPT, RPT)],
                        out.at[pl.ds(cid * NPAD + sid * RPT, RPT)])

    return k


def _edge_body(psrc_ref, pdst_ref, f11_ref, f21_ref, f12_ref, f22_ref,
               ew_ref, w2_ref):
    ev = psrc_ref[...] - pdst_ref[...]
    x = ev[:, 0:1]
    y = ev[:, 1:2]
    z = ev[:, 2:3]
    r = jnp.sqrt(x * x + y * y + z * z)
    inv = 1.0 / (r + 1e-9)
    ux, uy, uz = x * inv, y * inv, z * inv
    s5 = 5.0 ** 0.5
    s15 = 15.0 ** 0.5
    c1 = (35.0 / 8.0) ** 0.5
    c2 = 105.0 ** 0.5
    c3 = (21.0 / 8.0) ** 0.5
    c4 = (7.0 ** 0.5) / 2.0
    one = jnp.ones_like(ux)
    sh = jnp.concatenate([
        one,
        SQRT3 * ux, SQRT3 * uy, SQRT3 * uz,
        s15 * ux * uy,
        s15 * uy * uz,
        0.5 * s5 * (3.0 * uz * uz - 1.0),
        s15 * ux * uz,
        0.5 * s15 * (ux * ux - uy * uy),
        c1 * uy * (3.0 * ux * ux - uy * uy),
        c2 * ux * uy * uz,
        c3 * uy * (5.0 * uz * uz - 1.0),
        c4 * uz * (5.0 * uz * uz - 3.0),
        c3 * ux * (5.0 * uz * uz - 1.0),
        0.5 * c2 * uz * (ux * ux - uy * uy),
        c1 * ux * (ux * ux - 3.0 * uy * uy),
    ], axis=1)
    coli = lax.broadcasted_iota(jnp.int32, (BE, 16), 1)
    col = coli.astype(jnp.float32)
    d = (r - (1.0 + 0.5 * col)) * 2.0
    embp = jnp.where(coli < 3, jnp.exp(-d * d) * SQRT3, 0.0)
    h1 = jnp.maximum(jnp.dot(embp, f11_ref[...],
                             preferred_element_type=jnp.float32), 0.0)
    w1 = jnp.dot(h1, f21_ref[...],
                 preferred_element_type=jnp.float32) * (1.0 / (SQRT3 * 16.0))
    h2 = jnp.maximum(jnp.dot(embp, f12_ref[...],
                             preferred_element_type=jnp.float32), 0.0)
    w2_ref[...] = jnp.dot(h2, f22_ref[...],
                          preferred_element_type=jnp.float32) * (1.0 / (SQRT3 * 16.0))
    ew_ref[...] = jnp.concatenate(
        [sh, w1, jnp.zeros((BE, 96), jnp.float32)], axis=1)


def _msg1_body(x0_ref, ew_ref, wmix_ref, smix_ref, t1_ref, t2_ref):
    ew = ew_ref[...]
    xw = x0_ref[...][:, :16] * INV_SQRT_NEI * ew[:, 16:32]
    m1 = (jnp.dot(xw, wmix_ref[...], preferred_element_type=jnp.float32)
          * jnp.dot(ew[:, :16], smix_ref[...],
                    preferred_element_type=jnp.float32))
    t1_ref[...] = m1[:, :128]
    t2_ref[...] = jnp.concatenate(
        [m1[:, 128:], jnp.zeros((BE, 96), jnp.float32)], axis=1)


def _msg2_body(x1_ref, w2_ref, ew_ref, wmix_ref, smix_ref, m_ref):
    xw = x1_ref[...] * w2_ref[...]
    m2 = (jnp.dot(xw, wmix_ref[...], preferred_element_type=jnp.float32)
          * jnp.dot(ew_ref[...][:, :16], smix_ref[...],
                    preferred_element_type=jnp.float32))
    m_ref[...] = jnp.concatenate(
        [m2, jnp.zeros((BE, 112), jnp.float32)], axis=1)


def _add2_body(a_ref, b_ref, o_ref):
    o_ref[...] = a_ref[...] + b_ref[...]


def _gate_body(y1a_ref, y1b_ref, y2a_ref, y2b_ref, x1_ref):
    p = (y1a_ref[...] + y1b_ref[...]) * INV_SQRT_NEI
    q = (y2a_ref[...] + y2b_ref[...]) * INV_SQRT_NEI
    xx = jnp.concatenate([p, q[:, :32]], axis=1)
    scalars = jnp.concatenate(
        [jnp.maximum(xx[:, :16], 0.0), jnp.abs(xx[:, 16:32])], axis=1)
    gates = jnp.concatenate([
        jnp.maximum(xx[:, 32:40], 0.0), jnp.tanh(xx[:, 40:48]),
        jnp.maximum(xx[:, 48:56], 0.0), jnp.tanh(xx[:, 56:64]),
    ], axis=1)
    g_row = lax.broadcasted_iota(jnp.int32, (32, 96), 0)
    j_col = lax.broadcasted_iota(jnp.int32, (32, 96), 1)
    expand = jnp.where(g_row == j_col // 3, 1.0, 0.0)
    gates96 = jnp.dot(gates, expand, preferred_element_type=jnp.float32)
    x1_ref[...] = jnp.concatenate([scalars, xx[:, 64:160] * gates96], axis=1)


def _graph_body(ya_ref, yb_ref, batch_ref, out_ref):
    i = pl.program_id(0)

    @pl.when(i == 0)
    def _():
        out_ref[...] = jnp.zeros_like(out_ref)

    y = (ya_ref[0] + yb_ref[0]) * (INV_SQRT_NEI * 0.5)
    b = batch_ref[...]
    gcol = lax.broadcasted_iota(jnp.int32, (b.shape[0], NG), 1).astype(jnp.float32)
    onehot = jnp.where(b == gcol, 1.0, 0.0)
    out_ref[...] += lax.dot_general(onehot, y, (((0,), (0,)), ((), ())),
                                    preferred_element_type=jnp.float32)


def _wfull(a):
    return pl.BlockSpec(a.shape, lambda i: (0,) * a.ndim)


def _edge_precompute(psrc, pdst, f11, f21, f12, f22):
    grid = EPAD // BE
    eb = pl.BlockSpec((BE, 128), lambda i: (i, 0))
    return pl.pallas_call(
        _edge_body,
        grid=(grid,),
        in_specs=[eb, eb, _wfull(f11), _wfull(f21), _wfull(f12), _wfull(f22)],
        out_specs=[eb, eb],
        out_shape=[jax.ShapeDtypeStruct((EPAD, 128), jnp.float32),
                   jax.ShapeDtypeStruct((EPAD, 128), jnp.float32)],
    )(psrc, pdst, f11, f21, f12, f22)


def _messages1(x0src, ew, wmix, smix):
    grid = EPAD // BE
    eb = pl.BlockSpec((BE, 128), lambda i: (i, 0))
    return pl.pallas_call(
        _msg1_body,
        grid=(grid,),
        in_specs=[eb, eb, _wfull(wmix), _wfull(smix)],
        out_specs=[eb, eb],
        out_shape=[jax.ShapeDtypeStruct((EPAD, 128), jnp.float32),
                   jax.ShapeDtypeStruct((EPAD, 128), jnp.float32)],
    )(x0src, ew, wmix, smix)


def _messages2(x1src, w2, ew, wmix, smix):
    grid = EPAD // BE
    eb = pl.BlockSpec((BE, 128), lambda i: (i, 0))
    return pl.pallas_call(
        _msg2_body,
        grid=(grid,),
        in_specs=[eb, eb, eb, _wfull(wmix), _wfull(smix)],
        out_specs=eb,
        out_shape=jax.ShapeDtypeStruct((EPAD, 128), jnp.float32),
    )(x1src, w2, ew, wmix, smix)


def _add2(pair):
    BN = 1024
    grid = NPAD // BN
    return pl.pallas_call(
        _add2_body,
        grid=(grid,),
        in_specs=[pl.BlockSpec((BN, 128), lambda i: (i, 0)),
                  pl.BlockSpec((BN, 128), lambda i: (i + NPAD // BN, 0))],
        out_specs=pl.BlockSpec((BN, 128), lambda i: (i, 0)),
        out_shape=jax.ShapeDtypeStruct((NPAD, 128), jnp.float32),
    )(pair, pair)


def _gate(y1p, y2p):
    BN = 1024
    grid = NPAD // BN
    lo = pl.BlockSpec((BN, 128), lambda i: (i, 0))
    hi = pl.BlockSpec((BN, 128), lambda i: (i + NPAD // BN, 0))
    return pl.pallas_call(
        _gate_body,
        grid=(grid,),
        in_specs=[lo, hi, lo, hi],
        out_specs=lo,
        out_shape=jax.ShapeDtypeStruct((NPAD, 128), jnp.float32),
    )(y1p, y1p, y2p, y2p)


def _graph_reduce(y2pair, batch_f):
    BN = 2000
    grid = N // BN
    return pl.pallas_call(
        _graph_body,
        grid=(grid,),
        in_specs=[pl.BlockSpec((1, BN, 128), lambda i: (0, i, 0)),
                  pl.BlockSpec((1, BN, 128), lambda i: (1, i, 0)),
                  pl.BlockSpec((BN, 1), lambda i: (i, 0))],
        out_specs=pl.BlockSpec((NG, 128), lambda i: (0, 0)),
        out_shape=jax.ShapeDtypeStruct((NG, 128), jnp.float32),
    )(y2pair, y2pair, batch_f)


def kernel(pos, batch, edge_src, edge_dst, fc1_1, fc2_1, Wmix1, Smix1,
           fc1_2, fc2_2, Wmix2, Smix2):
    f32 = jnp.float32
    es = edge_src.astype(jnp.int32)
    ed = edge_dst.astype(jnp.int32)
    padn = EPAD - E
    es_g = jnp.concatenate([es, jnp.zeros((padn,), jnp.int32)])
    ed_g = jnp.concatenate([ed, jnp.zeros((padn,), jnp.int32)])
    ed_s = jnp.concatenate([ed, jnp.full((padn,), N, jnp.int32)])
    i_src = es_g.reshape(NW, CPW, CHUNK)
    i_dst = ed_g.reshape(NW, CPW, CHUNK)
    i_dst_s = ed_s.reshape(NW, CPW, CHUNK)

    pos128 = jnp.pad(pos.astype(f32), ((0, NPAD - N), (0, 125)))
    f11 = jnp.pad(fc1_1.astype(f32), ((0, 13), (0, 0)))
    f12 = jnp.pad(fc1_2.astype(f32), ((0, 13), (0, 0)))
    wmix2p = jnp.pad(Wmix2.astype(f32), ((0, 0), (0, 9)))
    smix2p = jnp.pad(Smix2.astype(f32), ((0, 0), (0, 9)))
    batch_f = batch.astype(f32)[:, None]
    z128 = jnp.zeros((NPAD, 128), f32)

    psrc = _make_gather1(NPAD)(pos128, i_src)
    pdst = _make_gather1(NPAD)(pos128, i_dst)
    ew, w2 = _edge_precompute(psrc, pdst, f11, fc2_1.astype(f32), f12,
                              fc2_2.astype(f32))
    x0p = _make_scatter()(ew, i_dst_s, z128)
    x0c = _add2(x0p)
    x0src = _make_gather1(NPAD)(x0c, i_src)
    t1, t2 = _messages1(x0src, ew, Wmix1.astype(f32), Smix1.astype(f32))
    y1p = _make_scatter()(t1, i_dst_s, z128)
    y2p_ = _make_scatter()(t2, i_dst_s, z128)
    x1 = _gate(y1p, y2p_)
    x1src = _make_gather1(NPAD)(x1, i_src)
    m2 = _messages2(x1src, w2, ew, wmix2p, smix2p)
    y2p = _make_scatter()(m2, i_dst_s, z128)
    out = _graph_reduce(y2p.reshape(2, NPAD, 128), batch_f)
    return out[:, :7]


# speedup vs baseline: 3.1505x; 1.5150x over previous
import functools

import jax
import jax.numpy as jnp
import numpy as np
from jax import lax
from jax.experimental import pallas as pl
from jax.experimental.pallas import tpu as pltpu
from jax.experimental.pallas import tpu_sc as plsc

N = 10000
E = 160000
NG = 64
NPAD = 10240
CHUNK = 128
NW = 32
CPW = 40
EPAD = NW * CPW * CHUNK
BE = 2048
INV_SQRT_NEI = float(3.8 ** -0.5)
SQRT3 = float(3.0 ** 0.5)


def _mesh():
    return plsc.VectorSubcoreMesh(core_axis_name="c", subcore_axis_name="s")


NB = 4
NROUNDS = CPW // NB


def _make_gather1(T):
    NBG = 2
    TPS = T // 16
    scratch = ([pltpu.VMEM((CPW, CHUNK), jnp.int32)]
               + [pltpu.VMEM((CHUNK, 128), jnp.float32)] * NBG
               + [pltpu.SemaphoreType.DMA] * (2 * NBG)
               + [pltpu.VMEM_SHARED((T, 128), jnp.float32)])

    @functools.partial(
        pl.kernel, out_type=jax.ShapeDtypeStruct((EPAD, 128), jnp.float32),
        mesh=_mesh(), scratch_types=scratch,
    )
    def k(table, idx, out, i_v, *rs):
        bufs = rs[:NBG]
        gs = rs[NBG:2 * NBG]
        ws = rs[2 * NBG:3 * NBG]
        stab = rs[3 * NBG]
        sid = lax.axis_index("s")
        wid = sid * 2 + lax.axis_index("c")
        base = wid * (CPW * CHUNK)
        pltpu.sync_copy(table.at[pl.ds(sid * TPS, TPS)],
                        stab.at[pl.ds(sid * TPS, TPS)])
        pltpu.sync_copy(idx.at[wid], i_v)
        plsc.subcore_barrier()
        for b in range(NBG):
            pltpu.async_copy(stab.at[i_v.at[b]], bufs[b], gs[b])

        def rnd(r, c):
            for b in range(NBG):
                j = r * NBG + b
                pltpu.make_async_copy(stab.at[i_v.at[j]], bufs[b],
                                      gs[b]).wait()
                pltpu.async_copy(bufs[b],
                                 out.at[pl.ds(base + j * CHUNK, CHUNK)],
                                 ws[b])
            for b in range(NBG):
                jn = (r + 1) * NBG + b

                @pl.when(jn < CPW)
                def _():
                    pltpu.make_async_copy(
                        bufs[b], out.at[pl.ds(base, CHUNK)], ws[b]).wait()
                    pltpu.async_copy(stab.at[i_v.at[jn]], bufs[b], gs[b])
            return c

        lax.fori_loop(0, CPW // NBG, rnd, 0)
        for b in range(NBG):
            pltpu.make_async_copy(bufs[b], out.at[pl.ds(base, CHUNK)],
                                  ws[b]).wait()

    return k


def _make_scatter():
    RPT = NPAD // 16
    NBS = 2
    scratch = ([pltpu.VMEM((CPW, CHUNK), jnp.int32)]
               + [pltpu.VMEM((CHUNK, 128), jnp.float32)] * NBS
               + [pltpu.SemaphoreType.DMA] * (2 * NBS))

    @functools.partial(
        pl.kernel, out_type=jax.ShapeDtypeStruct((2 * NPAD, 128), jnp.float32),
        mesh=_mesh(),
        scratch_types=scratch + [pltpu.VMEM_SHARED((NPAD, 128), jnp.float32)],
    )
    def k(data, idx, zeros, out, i_v, *rs):
        bufs = rs[:NBS]
        ls = rs[NBS:2 * NBS]
        ss = rs[2 * NBS:3 * NBS]
        acc = rs[3 * NBS]
        cid = lax.axis_index("c")
        sid = lax.axis_index("s")
        wid = sid * 2 + cid
        base = wid * (CPW * CHUNK)
        pltpu.sync_copy(zeros.at[pl.ds(sid * RPT, RPT)],
                        acc.at[pl.ds(sid * RPT, RPT)])
        pltpu.sync_copy(idx.at[wid], i_v)
        plsc.subcore_barrier()
        for b in range(NBS):
            pltpu.async_copy(data.at[pl.ds(base + b * CHUNK, CHUNK)],
                             bufs[b], ls[b])

        def rnd(r, c):
            for b in range(NBS):
                j = r * NBS + b
                pltpu.make_async_copy(
                    data.at[pl.ds(base, CHUNK)], bufs[b], ls[b]).wait()
                pltpu.async_copy(bufs[b], acc.at[i_v.at[j]], ss[b], add=True)
            for b in range(NBS):
                jn = (r + 1) * NBS + b

                @pl.when(jn < CPW)
                def _():
                    pltpu.make_async_copy(bufs[b], acc.at[i_v.at[0]],
                                          ss[b]).wait()
                    pltpu.async_copy(data.at[pl.ds(base + jn * CHUNK, CHUNK)],
                                     bufs[b], ls[b])
            return c

        lax.fori_loop(0, CPW // NBS, rnd, 0)
        for b in range(NBS):
            pltpu.make_async_copy(bufs[b], acc.at[i_v.at[0]], ss[b]).wait()
        plsc.subcore_barrier()
        pltpu.sync_copy(acc.at[pl.ds(sid * RPT, RPT)],
                        out.at[pl.ds(cid * NPAD + sid * RPT, RPT)])

    return k


def _sh_constants():
    A = np.zeros((128, 128), np.float32)
    B = np.zeros((128, 128), np.float32)
    for i in range(4):
        for j in range(4):
            A[i, 4 * i + j] = 1.0
            B[j, 4 * i + j] = 1.0
    A2 = np.zeros((128, 128), np.float32)
    B2 = np.zeros((128, 128), np.float32)
    for k in range(16):
        for j in range(4):
            A2[k, 4 * k + j] = 1.0
            B2[j, 4 * k + j] = 1.0
    s5 = 5.0 ** 0.5
    s15 = 15.0 ** 0.5
    c1 = (35.0 / 8.0) ** 0.5
    c2 = 105.0 ** 0.5
    c3 = (21.0 / 8.0) ** 0.5
    c4 = (7.0 ** 0.5) / 2.0
    terms = {
        0: [((3, 3, 3), 1.0)],
        1: [((0, 3, 3), SQRT3)],
        2: [((1, 3, 3), SQRT3)],
        3: [((2, 3, 3), SQRT3)],
        4: [((0, 1, 3), s15)],
        5: [((1, 2, 3), s15)],
        6: [((2, 2, 3), 1.5 * s5), ((3, 3, 3), -0.5 * s5)],
        7: [((0, 2, 3), s15)],
        8: [((0, 0, 3), 0.5 * s15), ((1, 1, 3), -0.5 * s15)],
        9: [((0, 0, 1), 3 * c1), ((1, 1, 1), -c1)],
        10: [((0, 1, 2), c2)],
        11: [((1, 2, 2), 5 * c3), ((1, 3, 3), -c3)],
        12: [((2, 2, 2), 5 * c4), ((2, 3, 3), -3 * c4)],
        13: [((0, 2, 2), 5 * c3), ((0, 3, 3), -c3)],
        14: [((0, 0, 2), 0.5 * c2), ((1, 1, 2), -0.5 * c2)],
        15: [((0, 0, 0), c1), ((0, 1, 1), -3 * c1)],
    }
    D3 = np.zeros((128, 128), np.float32)
    for comp, ts in terms.items():
        for f, coeff in ts:
            D3[16 * f[0] + 4 * f[1] + f[2], comp] += coeff
    O3 = np.zeros((128, 128), np.float32)
    O3[:3, :] = 1.0
    return A, B, A2, B2, D3, O3


def _edge_body(psrc_ref, pdst_ref, f11_ref, f21_ref, f12_ref, f22_ref,
               a_ref, b_ref, a2_ref, b2_ref, d3_ref, o3_ref,
               ew_ref, w2_ref):
    f32 = jnp.float32

    def mm(a, b):
        return jnp.dot(a, b, preferred_element_type=f32)

    ev = psrc_ref[...] - pdst_ref[...]
    rb = jnp.sqrt(mm(ev * ev, o3_ref[...]))
    invb = 1.0 / (rb + 1e-9)
    coli = lax.broadcasted_iota(jnp.int32, (BE, 128), 1)
    q = jnp.where(coli < 3, ev, jnp.where(coli == 3, rb + 1e-9, 0.0)) * invb
    p2 = mm(q, a_ref[...]) * mm(q, b_ref[...])
    p3 = mm(p2, a2_ref[...]) * mm(q, b2_ref[...])
    d = (rb - (1.0 + 0.5 * coli.astype(f32))) * 2.0
    embq = jnp.where(coli < 3, jnp.exp(-d * d) * SQRT3, 0.0)
    h1 = jnp.maximum(mm(embq, f11_ref[...]), 0.0)
    h2 = jnp.maximum(mm(embq, f12_ref[...]), 0.0)
    ew_ref[...] = mm(p3, d3_ref[...]) + mm(h1, f21_ref[...])
    w2_ref[...] = mm(h2, f22_ref[...])


def _msg1_body(x0_ref, ew_ref, wmix_ref, smix_ref, t1_ref, t2_ref):
    ew = ew_ref[...]
    xw = x0_ref[...][:, :16] * INV_SQRT_NEI * ew[:, 16:32]
    m1 = (jnp.dot(xw, wmix_ref[...], preferred_element_type=jnp.float32)
          * jnp.dot(ew[:, :16], smix_ref[...],
                    preferred_element_type=jnp.float32))
    t1_ref[...] = m1[:, :128]
    t2_ref[...] = jnp.concatenate(
        [m1[:, 128:], jnp.zeros((BE, 96), jnp.float32)], axis=1)


def _msg2_body(x1_ref, w2_ref, ew_ref, wmix_ref, smix_ref, m_ref):
    xw = x1_ref[...] * w2_ref[...]
    m2 = (jnp.dot(xw, wmix_ref[...], preferred_element_type=jnp.float32)
          * jnp.dot(ew_ref[...][:, :16], smix_ref[...],
                    preferred_element_type=jnp.float32))
    m_ref[...] = jnp.concatenate(
        [m2, jnp.zeros((BE, 112), jnp.float32)], axis=1)


def _add2_body(a_ref, b_ref, o_ref):
    o_ref[...] = a_ref[...] + b_ref[...]


def _gate_body(y1a_ref, y1b_ref, y2a_ref, y2b_ref, x1_ref):
    p = (y1a_ref[...] + y1b_ref[...]) * INV_SQRT_NEI
    q = (y2a_ref[...] + y2b_ref[...]) * INV_SQRT_NEI
    xx = jnp.concatenate([p, q[:, :32]], axis=1)
    scalars = jnp.concatenate(
        [jnp.maximum(xx[:, :16], 0.0), jnp.abs(xx[:, 16:32])], axis=1)
    gates = jnp.concatenate([
        jnp.maximum(xx[:, 32:40], 0.0), jnp.tanh(xx[:, 40:48]),
        jnp.maximum(xx[:, 48:56], 0.0), jnp.tanh(xx[:, 56:64]),
    ], axis=1)
    g_row = lax.broadcasted_iota(jnp.int32, (32, 96), 0)
    j_col = lax.broadcasted_iota(jnp.int32, (32, 96), 1)
    expand = jnp.where(g_row == j_col // 3, 1.0, 0.0)
    gates96 = jnp.dot(gates, expand, preferred_element_type=jnp.float32)
    x1_ref[...] = jnp.concatenate([scalars, xx[:, 64:160] * gates96], axis=1)


def _graph_body(ya_ref, yb_ref, batch_ref, out_ref):
    i = pl.program_id(0)

    @pl.when(i == 0)
    def _():
        out_ref[...] = jnp.zeros_like(out_ref)

    y = (ya_ref[0] + yb_ref[0]) * (INV_SQRT_NEI * 0.5)
    b = batch_ref[...]
    gcol = lax.broadcasted_iota(jnp.int32, (b.shape[0], NG), 1).astype(jnp.float32)
    onehot = jnp.where(b == gcol, 1.0, 0.0)
    out_ref[...] += lax.dot_general(onehot, y, (((0,), (0,)), ((), ())),
                                    preferred_element_type=jnp.float32)


def _wfull(a):
    return pl.BlockSpec(a.shape, lambda i: (0,) * a.ndim)


def _edge_precompute(psrc, pdst, f11, f21, f12, f22, consts):
    grid = EPAD // BE
    eb = pl.BlockSpec((BE, 128), lambda i: (i, 0))
    ws = [f11, f21, f12, f22] + list(consts)
    return pl.pallas_call(
        _edge_body,
        grid=(grid,),
        in_specs=[eb, eb] + [_wfull(w) for w in ws],
        out_specs=[eb, eb],
        out_shape=[jax.ShapeDtypeStruct((EPAD, 128), jnp.float32),
                   jax.ShapeDtypeStruct((EPAD, 128), jnp.float32)],
    )(psrc, pdst, *ws)


def _messages1(x0src, ew, wmix, smix):
    grid = EPAD // BE
    eb = pl.BlockSpec((BE, 128), lambda i: (i, 0))
    return pl.pallas_call(
        _msg1_body,
        grid=(grid,),
        in_specs=[eb, eb, _wfull(wmix), _wfull(smix)],
        out_specs=[eb, eb],
        out_shape=[jax.ShapeDtypeStruct((EPAD, 128), jnp.float32),
                   jax.ShapeDtypeStruct((EPAD, 128), jnp.float32)],
    )(x0src, ew, wmix, smix)


def _messages2(x1src, w2, ew, wmix, smix):
    grid = EPAD // BE
    eb = pl.BlockSpec((BE, 128), lambda i: (i, 0))
    return pl.pallas_call(
        _msg2_body,
        grid=(grid,),
        in_specs=[eb, eb, eb, _wfull(wmix), _wfull(smix)],
        out_specs=eb,
        out_shape=jax.ShapeDtypeStruct((EPAD, 128), jnp.float32),
    )(x1src, w2, ew, wmix, smix)


def _add2(pair):
    BN = 1024
    grid = NPAD // BN
    return pl.pallas_call(
        _add2_body,
        grid=(grid,),
        in_specs=[pl.BlockSpec((BN, 128), lambda i: (i, 0)),
                  pl.BlockSpec((BN, 128), lambda i: (i + NPAD // BN, 0))],
        out_specs=pl.BlockSpec((BN, 128), lambda i: (i, 0)),
        out_shape=jax.ShapeDtypeStruct((NPAD, 128), jnp.float32),
    )(pair, pair)


def _gate(y1p, y2p):
    BN = 1024
    grid = NPAD // BN
    lo = pl.BlockSpec((BN, 128), lambda i: (i, 0))
    hi = pl.BlockSpec((BN, 128), lambda i: (i + NPAD // BN, 0))
    return pl.pallas_call(
        _gate_body,
        grid=(grid,),
        in_specs=[lo, hi, lo, hi],
        out_specs=lo,
        out_shape=jax.ShapeDtypeStruct((NPAD, 128), jnp.float32),
    )(y1p, y1p, y2p, y2p)


def _graph_reduce(y2pair, batch_f):
    BN = 2000
    grid = N // BN
    return pl.pallas_call(
        _graph_body,
        grid=(grid,),
        in_specs=[pl.BlockSpec((1, BN, 128), lambda i: (0, i, 0)),
                  pl.BlockSpec((1, BN, 128), lambda i: (1, i, 0)),
                  pl.BlockSpec((BN, 1), lambda i: (i, 0))],
        out_specs=pl.BlockSpec((NG, 128), lambda i: (0, 0)),
        out_shape=jax.ShapeDtypeStruct((NG, 128), jnp.float32),
    )(y2pair, y2pair, batch_f)


def kernel(pos, batch, edge_src, edge_dst, fc1_1, fc2_1, Wmix1, Smix1,
           fc1_2, fc2_2, Wmix2, Smix2):
    f32 = jnp.float32
    es = edge_src.astype(jnp.int32)
    ed = edge_dst.astype(jnp.int32)
    padn = EPAD - E
    es_g = jnp.concatenate([es, jnp.zeros((padn,), jnp.int32)])
    ed_g = jnp.concatenate([ed, jnp.zeros((padn,), jnp.int32)])
    ed_s = jnp.concatenate([ed, jnp.full((padn,), N, jnp.int32)])
    i_src = es_g.reshape(NW, CPW, CHUNK)
    i_dst = ed_g.reshape(NW, CPW, CHUNK)
    i_dst_s = ed_s.reshape(NW, CPW, CHUNK)

    pos128 = jnp.pad(pos.astype(f32), ((0, NPAD - N), (0, 125)))
    mlp_s = 1.0 / (SQRT3 * 16.0)
    f11 = jnp.pad(fc1_1.astype(f32), ((0, 125), (0, 0)))
    f12 = jnp.pad(fc1_2.astype(f32), ((0, 125), (0, 0)))
    f21 = jnp.zeros((256, 128), f32).at[:, 16:32].set(fc2_1.astype(f32) * mlp_s)
    f22 = fc2_2.astype(f32) * mlp_s
    consts = [jnp.asarray(c) for c in _sh_constants()]
    wmix2p = jnp.pad(Wmix2.astype(f32), ((0, 0), (0, 9)))
    smix2p = jnp.pad(Smix2.astype(f32), ((0, 0), (0, 9)))
    batch_f = batch.astype(f32)[:, None]
    z128 = jnp.zeros((NPAD, 128), f32)

    psrc = _make_gather1(NPAD)(pos128, i_src)
    pdst = _make_gather1(NPAD)(pos128, i_dst)
    ew, w2 = _edge_precompute(psrc, pdst, f11, f21, f12, f22, consts)
    x0p = _make_scatter()(ew, i_dst_s, z128)
    x0c = _add2(x0p)
    x0src = _make_gather1(NPAD)(x0c, i_src)
    t1, t2 = _messages1(x0src, ew, Wmix1.astype(f32), Smix1.astype(f32))
    y1p = _make_scatter()(t1, i_dst_s, z128)
    y2p_ = _make_scatter()(t2, i_dst_s, z128)
    x1 = _gate(y1p, y2p_)
    x1src = _make_gather1(NPAD)(x1, i_src)
    m2 = _messages2(x1src, w2, ew, wmix2p, smix2p)
    y2p = _make_scatter()(m2, i_dst_s, z128)
    out = _graph_reduce(y2p.reshape(2, NPAD, 128), batch_f)
    return out[:, :7]
